# baseline (speedup 1.0000x reference)
import functools

import jax
import jax.numpy as jnp
from jax import lax
from jax.experimental import pallas as pl
from jax.experimental.pallas import tpu as pltpu
from jax.experimental.pallas import tpu_sc as plsc

V_CHUNK = 400
F_CHUNK = 128
RW = 8


def _sc_accumulate(vertices, faces_t, zeros_row, B, V, FPAD):
    info = plsc.get_sparse_core_info()
    NC, NS, L = info.num_cores, info.num_subcores, info.num_lanes
    BH = B // NC
    faces_per_tile = FPAD // NS
    n_fchunks = faces_per_tile // F_CHUNK
    nv_chunks = V // V_CHUNK
    mesh = plsc.VectorSubcoreMesh(core_axis_name="c", subcore_axis_name="s")

    @functools.partial(
        pl.kernel,
        out_type=jax.ShapeDtypeStruct((NC, V, RW), jnp.float32),
        mesh=mesh,
        compiler_params=pltpu.CompilerParams(
            needs_layout_passes=False, use_tc_tiling_on_sc=False),
        scratch_types=[
            pltpu.VMEM_SHARED((V, RW), jnp.float32),
            pltpu.VMEM_SHARED((V, RW), jnp.float32),
            pltpu.VMEM((V_CHUNK, 3), jnp.float32),
            pltpu.VMEM((V_CHUNK, RW), jnp.float32),
            pltpu.VMEM((V_CHUNK, RW), jnp.float32),
            pltpu.VMEM((F_CHUNK, 3), jnp.int32),
            pltpu.VMEM((3, F_CHUNK), jnp.int32),
            pltpu.VMEM((3, F_CHUNK, RW), jnp.float32),
            pltpu.VMEM((F_CHUNK, RW), jnp.float32),
        ],
    )
    def sc_kernel(verts_hbm, faces_hbm, zeros_hbm, out_hbm,
                  uw_s, acc_s, vbuf, uwbuf, zbuf, fbuf, idxbuf, gbuf, sbuf):
        cid = lax.axis_index("c")
        sid = lax.axis_index("s")
        iota = lax.iota(jnp.int32, L)

        def cfull(val):
            return jnp.full((L,), val, jnp.int32)

        pltpu.sync_copy(zeros_hbm, zbuf)

        n_mine = jnp.where(sid < (nv_chunks % NS), nv_chunks // NS + 1,
                           nv_chunks // NS).astype(jnp.int32)

        def build_body(i, _):
            c = sid + i * NS
            v0 = c * V_CHUNK
            for bb in range(BH):
                b = cid * BH + bb
                pltpu.sync_copy(verts_hbm.at[b, pl.ds(v0, V_CHUNK), :], vbuf)

                def grp(j, _):
                    r = iota + j * L
                    x = plsc.load_gather(vbuf, [r, cfull(0)])
                    y = plsc.load_gather(vbuf, [r, cfull(1)])
                    z = plsc.load_gather(vbuf, [r, cfull(2)])
                    plsc.store_scatter(uwbuf, [r, cfull(bb)], y - x)
                    plsc.store_scatter(uwbuf, [r, cfull(BH + bb)], z - x)
                    return 0
                lax.fori_loop(0, V_CHUNK // L, grp, 0)
            pltpu.sync_copy(uwbuf, uw_s.at[pl.ds(v0, V_CHUNK), :])
            pltpu.sync_copy(zbuf, acc_s.at[pl.ds(v0, V_CHUNK), :])
            return 0
        lax.fori_loop(0, n_mine, build_body, 0)

        plsc.subcore_barrier()

        def face_body(i, _):
            f0 = sid * faces_per_tile + i * F_CHUNK
            pltpu.sync_copy(faces_hbm.at[pl.ds(f0, F_CHUNK), :], fbuf)
            for j in range(F_CHUNK // L):
                r = iota + j * L
                for k in range(3):
                    col = plsc.load_gather(fbuf, [r, cfull(k)])
                    plsc.store_scatter(idxbuf, [cfull(k), r], col)
            for k in range(3):
                pltpu.sync_copy(uw_s.at[idxbuf.at[k]], gbuf.at[k])
            for j in range(F_CHUNK // L):
                r = iota + j * L
                for bb in range(BH):
                    a0 = plsc.load_gather(gbuf, [cfull(0), r, cfull(bb)])
                    a1 = plsc.load_gather(gbuf, [cfull(1), r, cfull(bb)])
                    a2 = plsc.load_gather(gbuf, [cfull(2), r, cfull(bb)])
                    c0 = plsc.load_gather(gbuf, [cfull(0), r, cfull(BH + bb)])
                    c1 = plsc.load_gather(gbuf, [cfull(1), r, cfull(BH + bb)])
                    c2 = plsc.load_gather(gbuf, [cfull(2), r, cfull(BH + bb)])
                    plsc.store_scatter(sbuf, [r, cfull(bb * 3 + 0)],
                                       a1 * c2 - a2 * c1)
                    plsc.store_scatter(sbuf, [r, cfull(bb * 3 + 1)],
                                       a2 * c0 - a0 * c2)
                    plsc.store_scatter(sbuf, [r, cfull(bb * 3 + 2)],
                                       a0 * c1 - a1 * c0)
                plsc.store_scatter(sbuf, [r, cfull(6)], jnp.zeros((L,), jnp.float32))
                plsc.store_scatter(sbuf, [r, cfull(7)], jnp.zeros((L,), jnp.float32))
            for k in range(3):
                pltpu.sync_copy(sbuf, acc_s.at[idxbuf.at[k]], add=True)
            return 0
        lax.fori_loop(0, n_fchunks, face_body, 0)

        plsc.subcore_barrier()

        def wo_body(i, _):
            c = sid + i * NS
            v0 = c * V_CHUNK
            pltpu.sync_copy(acc_s.at[pl.ds(v0, V_CHUNK), :], zbuf)
            pltpu.sync_copy(zbuf, out_hbm.at[cid, pl.ds(v0, V_CHUNK), :])
            return 0
        lax.fori_loop(0, n_mine, wo_body, 0)

    return sc_kernel(vertices, faces_t, zeros_row)


def _tc_normalize(partials, B, V):
    VB = 800
    NC = partials.shape[0]
    BH = B // NC

    def body(p_ref, o_ref):
        for c in range(NC):
            s = p_ref[c]
            for bb in range(BH):
                sl = s[:, bb * 3:(bb + 1) * 3]
                nrm = jnp.sqrt(jnp.sum(sl * sl, axis=1, keepdims=True))
                o_ref[c * BH + bb] = sl / jnp.maximum(nrm, 1e-6)

    return pl.pallas_call(
        body,
        grid=(V // VB,),
        in_specs=[pl.BlockSpec((NC, VB, RW), lambda i: (0, i, 0))],
        out_specs=pl.BlockSpec((B, VB, 3), lambda i: (0, i, 0)),
        out_shape=jax.ShapeDtypeStruct((B, V, 3), jnp.float32),
    )(partials)


def kernel(vertices, faces):
    faces = jnp.squeeze(faces).astype(jnp.int32)
    B, V, _ = vertices.shape
    F = faces.shape[0]
    NS = 16
    FPAD = -(-F // (NS * F_CHUNK)) * (NS * F_CHUNK)
    faces_pad = jnp.concatenate(
        [faces, jnp.zeros((FPAD - F, 3), jnp.int32)], axis=0)
    zeros_row = jnp.zeros((V_CHUNK, RW), jnp.float32)
    partials = _sc_accumulate(vertices, faces_pad, zeros_row, B, V, FPAD)
    return _tc_normalize(partials, B, V)


# speedup vs baseline: 1.0599x; 1.0599x over previous
import functools

import jax
import jax.numpy as jnp
from jax import lax
from jax.experimental import pallas as pl
from jax.experimental.pallas import tpu as pltpu
from jax.experimental.pallas import tpu_sc as plsc

V_CHUNK = 400
F_CHUNK = 128
RW = 8


def _rsqrt_newton(x):
    xu = plsc.bitcast(x, jnp.uint32)
    y = plsc.bitcast(jnp.uint32(0x5F3759DF) - (xu >> jnp.uint32(1)), jnp.float32)
    half = x * 0.5
    for _ in range(3):
        y = y * (1.5 - half * y * y)
    return y


def kernel(vertices, faces):
    faces = jnp.squeeze(faces).astype(jnp.int32)
    B, V, _ = vertices.shape
    F = faces.shape[0]

    info = plsc.get_sparse_core_info()
    NC, NS, L = info.num_cores, info.num_subcores, info.num_lanes
    BH = B // NC
    n_full = F // F_CHUNK
    f_tail = F % F_CHUNK
    assert f_tail % L == 0
    nv_chunks = V // V_CHUNK
    mesh = plsc.VectorSubcoreMesh(core_axis_name="c", subcore_axis_name="s")

    @functools.partial(
        pl.kernel,
        out_type=jax.ShapeDtypeStruct((B, V, 3), jnp.float32),
        mesh=mesh,
        compiler_params=pltpu.CompilerParams(
            needs_layout_passes=False, use_tc_tiling_on_sc=False),
        scratch_types=[
            pltpu.VMEM_SHARED((V, RW), jnp.float32),
            pltpu.VMEM_SHARED((V, RW), jnp.float32),
            pltpu.VMEM((V_CHUNK, 3), jnp.float32),
            pltpu.VMEM((V_CHUNK, RW), jnp.float32),
            pltpu.VMEM((V_CHUNK, RW), jnp.float32),
            pltpu.VMEM((V_CHUNK, 3), jnp.float32),
            pltpu.VMEM((F_CHUNK, 3), jnp.int32),
            pltpu.VMEM((3, F_CHUNK), jnp.int32),
            pltpu.VMEM((3, F_CHUNK, RW), jnp.float32),
            pltpu.VMEM((F_CHUNK, RW), jnp.float32),
        ],
    )
    def sc_kernel(verts_hbm, faces_hbm, out_hbm,
                  uw_s, acc_s, vbuf, uwbuf, zbuf, obuf, fbuf, idxbuf, gbuf,
                  sbuf):
        cid = lax.axis_index("c")
        sid = lax.axis_index("s")
        iota = lax.iota(jnp.int32, L)
        zvec = jnp.zeros((L,), jnp.float32)

        def cfull(val):
            return jnp.full((L,), val, jnp.int32)

        for j in range(V_CHUNK // L):
            r = iota + j * L
            for col in range(RW):
                plsc.store_scatter(zbuf, [r, cfull(col)], zvec)

        n_mine = jnp.where(sid < (nv_chunks % NS), nv_chunks // NS + 1,
                           nv_chunks // NS).astype(jnp.int32)

        def build_body(i, _):
            c = sid + i * NS
            v0 = c * V_CHUNK
            for bb in range(BH):
                b = cid * BH + bb
                pltpu.sync_copy(verts_hbm.at[b, pl.ds(v0, V_CHUNK), :], vbuf)

                def grp(j, _):
                    r = iota + j * L
                    x = plsc.load_gather(vbuf, [r, cfull(0)])
                    y = plsc.load_gather(vbuf, [r, cfull(1)])
                    z = plsc.load_gather(vbuf, [r, cfull(2)])
                    plsc.store_scatter(uwbuf, [r, cfull(bb)], y - x)
                    plsc.store_scatter(uwbuf, [r, cfull(BH + bb)], z - x)
                    return 0
                lax.fori_loop(0, V_CHUNK // L, grp, 0)
            pltpu.sync_copy(uwbuf, uw_s.at[pl.ds(v0, V_CHUNK), :])
            pltpu.sync_copy(zbuf, acc_s.at[pl.ds(v0, V_CHUNK), :])
            return 0
        lax.fori_loop(0, n_mine, build_body, 0)

        plsc.subcore_barrier()

        def do_chunk(f0, n):
            ngrp = n // L
            pltpu.sync_copy(faces_hbm.at[pl.ds(f0, n), :],
                            fbuf.at[pl.ds(0, n), :])
            for j in range(ngrp):
                r = iota + j * L
                for k in range(3):
                    col = plsc.load_gather(fbuf, [r, cfull(k)])
                    plsc.store_scatter(idxbuf, [cfull(k), r], col)
            for k in range(3):
                pltpu.sync_copy(uw_s.at[idxbuf.at[k, pl.ds(0, n)]],
                                gbuf.at[k, pl.ds(0, n), :])
            for j in range(ngrp):
                r = iota + j * L
                for bb in range(BH):
                    a0 = plsc.load_gather(gbuf, [cfull(0), r, cfull(bb)])
                    a1 = plsc.load_gather(gbuf, [cfull(1), r, cfull(bb)])
                    a2 = plsc.load_gather(gbuf, [cfull(2), r, cfull(bb)])
                    c0 = plsc.load_gather(gbuf, [cfull(0), r, cfull(BH + bb)])
                    c1 = plsc.load_gather(gbuf, [cfull(1), r, cfull(BH + bb)])
                    c2 = plsc.load_gather(gbuf, [cfull(2), r, cfull(BH + bb)])
                    plsc.store_scatter(sbuf, [r, cfull(bb * 3 + 0)],
                                       a1 * c2 - a2 * c1)
                    plsc.store_scatter(sbuf, [r, cfull(bb * 3 + 1)],
                                       a2 * c0 - a0 * c2)
                    plsc.store_scatter(sbuf, [r, cfull(bb * 3 + 2)],
                                       a0 * c1 - a1 * c0)
                plsc.store_scatter(sbuf, [r, cfull(6)], zvec)
                plsc.store_scatter(sbuf, [r, cfull(7)], zvec)
            for k in range(3):
                pltpu.sync_copy(sbuf.at[pl.ds(0, n), :],
                                acc_s.at[idxbuf.at[k, pl.ds(0, n)]], add=True)

        n_mine_f = jnp.where(sid < (n_full % NS), n_full // NS + 1,
                             n_full // NS).astype(jnp.int32)

        def face_body(i, _):
            do_chunk((sid + i * NS) * F_CHUNK, F_CHUNK)
            return 0
        lax.fori_loop(0, n_mine_f, face_body, 0)

        if f_tail:
            @pl.when(sid == NS - 1)
            def _():
                do_chunk(n_full * F_CHUNK, f_tail)

        plsc.subcore_barrier()

        def wo_body(i, _):
            c = sid + i * NS
            v0 = c * V_CHUNK
            pltpu.sync_copy(acc_s.at[pl.ds(v0, V_CHUNK), :], zbuf)
            for bb in range(BH):
                b = cid * BH + bb
                for j in range(V_CHUNK // L):
                    r = iota + j * L
                    n0 = plsc.load_gather(zbuf, [r, cfull(bb * 3 + 0)])
                    n1 = plsc.load_gather(zbuf, [r, cfull(bb * 3 + 1)])
                    n2 = plsc.load_gather(zbuf, [r, cfull(bb * 3 + 2)])
                    s2 = n0 * n0 + n1 * n1 + n2 * n2
                    nrm = s2 * _rsqrt_newton(s2)
                    rec = 1.0 / jnp.maximum(nrm, 1e-6)
                    plsc.store_scatter(obuf, [r, cfull(0)], n0 * rec)
                    plsc.store_scatter(obuf, [r, cfull(1)], n1 * rec)
                    plsc.store_scatter(obuf, [r, cfull(2)], n2 * rec)
                pltpu.sync_copy(obuf, out_hbm.at[b, pl.ds(v0, V_CHUNK), :])
            return 0
        lax.fori_loop(0, n_mine, wo_body, 0)

    return sc_kernel(vertices, faces)


# speedup vs baseline: 4.1649x; 3.9294x over previous
import functools

import jax
import jax.numpy as jnp
from jax import lax
from jax.experimental import pallas as pl
from jax.experimental.pallas import tpu as pltpu
from jax.experimental.pallas import tpu_sc as plsc

V_CHUNK = 400
F_CHUNK = 128
RW = 8


def _rsqrt_newton(x):
    xu = plsc.bitcast(x, jnp.uint32)
    y = plsc.bitcast(jnp.uint32(0x5F3759DF) - (xu >> jnp.uint32(1)), jnp.float32)
    half = x * 0.5
    for _ in range(3):
        y = y * (1.5 - half * y * y)
    return y


def kernel(vertices, faces):
    faces = jnp.squeeze(faces).astype(jnp.int32)
    B, V, _ = vertices.shape
    F = faces.shape[0]
    vt = jnp.transpose(vertices, (2, 0, 1))
    ft = faces.T

    info = plsc.get_sparse_core_info()
    NC, NS, L = info.num_cores, info.num_subcores, info.num_lanes
    BH = B // NC
    n_full = F // F_CHUNK
    f_tail = F % F_CHUNK
    assert f_tail % L == 0
    nv_chunks = V // V_CHUNK
    mesh = plsc.VectorSubcoreMesh(core_axis_name="c", subcore_axis_name="s")

    @functools.partial(
        pl.kernel,
        out_type=jax.ShapeDtypeStruct((3, B, V), jnp.float32),
        mesh=mesh,
        compiler_params=pltpu.CompilerParams(
            needs_layout_passes=False, use_tc_tiling_on_sc=False),
        scratch_types=[
            pltpu.VMEM_SHARED((V, RW), jnp.float32),
            pltpu.VMEM_SHARED((V, RW), jnp.float32),
            pltpu.VMEM((3, V_CHUNK), jnp.float32),
            pltpu.VMEM((V_CHUNK, RW), jnp.float32),
            pltpu.VMEM((V_CHUNK, RW), jnp.float32),
            pltpu.VMEM((3, V_CHUNK), jnp.float32),
            pltpu.VMEM((3, F_CHUNK), jnp.int32),
            pltpu.VMEM((3, F_CHUNK, RW), jnp.float32),
            pltpu.VMEM((F_CHUNK, RW), jnp.float32),
        ],
    )
    def sc_kernel(verts_hbm, faces_hbm, out_hbm,
                  uw_s, acc_s, vbuf, uwbuf, zbuf, obuf, idxbuf, gbuf, sbuf):
        cid = lax.axis_index("c")
        sid = lax.axis_index("s")
        iota = lax.iota(jnp.int32, L)
        zvec = jnp.zeros((L,), jnp.float32)

        def cfull(val):
            return jnp.full((L,), val, jnp.int32)

        for j in range(V_CHUNK // L):
            r = iota + j * L
            for col in range(RW):
                plsc.store_scatter(zbuf, [r, cfull(col)], zvec)

        n_mine = jnp.where(sid < (nv_chunks % NS), nv_chunks // NS + 1,
                           nv_chunks // NS).astype(jnp.int32)

        def build_body(i, _):
            c = sid + i * NS
            v0 = c * V_CHUNK
            for bb in range(BH):
                b = cid * BH + bb
                pltpu.sync_copy(verts_hbm.at[:, b, pl.ds(v0, V_CHUNK)], vbuf)

                def grp(j, _):
                    sl = pl.ds(j * L, L)
                    r = iota + j * L
                    x = vbuf[0, sl]
                    y = vbuf[1, sl]
                    z = vbuf[2, sl]
                    plsc.store_scatter(uwbuf, [r, cfull(bb)], y - x)
                    plsc.store_scatter(uwbuf, [r, cfull(BH + bb)], z - x)
                    return 0
                lax.fori_loop(0, V_CHUNK // L, grp, 0)
            pltpu.sync_copy(uwbuf, uw_s.at[pl.ds(v0, V_CHUNK), :])
            pltpu.sync_copy(zbuf, acc_s.at[pl.ds(v0, V_CHUNK), :])
            return 0
        lax.fori_loop(0, n_mine, build_body, 0)

        plsc.subcore_barrier()

        def do_chunk(f0, n):
            ngrp = n // L
            pltpu.sync_copy(faces_hbm.at[:, pl.ds(f0, n)],
                            idxbuf.at[:, pl.ds(0, n)])
            for k in range(3):
                pltpu.sync_copy(uw_s.at[idxbuf.at[k, pl.ds(0, n)]],
                                gbuf.at[k, pl.ds(0, n), :])
            for j in range(ngrp):
                r = iota + j * L
                for bb in range(BH):
                    a0 = plsc.load_gather(gbuf, [cfull(0), r, cfull(bb)])
                    a1 = plsc.load_gather(gbuf, [cfull(1), r, cfull(bb)])
                    a2 = plsc.load_gather(gbuf, [cfull(2), r, cfull(bb)])
                    c0 = plsc.load_gather(gbuf, [cfull(0), r, cfull(BH + bb)])
                    c1 = plsc.load_gather(gbuf, [cfull(1), r, cfull(BH + bb)])
                    c2 = plsc.load_gather(gbuf, [cfull(2), r, cfull(BH + bb)])
                    plsc.store_scatter(sbuf, [r, cfull(bb * 3 + 0)],
                                       a1 * c2 - a2 * c1)
                    plsc.store_scatter(sbuf, [r, cfull(bb * 3 + 1)],
                                       a2 * c0 - a0 * c2)
                    plsc.store_scatter(sbuf, [r, cfull(bb * 3 + 2)],
                                       a0 * c1 - a1 * c0)
                plsc.store_scatter(sbuf, [r, cfull(6)], zvec)
                plsc.store_scatter(sbuf, [r, cfull(7)], zvec)
            for k in range(3):
                pltpu.sync_copy(sbuf.at[pl.ds(0, n), :],
                                acc_s.at[idxbuf.at[k, pl.ds(0, n)]], add=True)

        n_mine_f = jnp.where(sid < (n_full % NS), n_full // NS + 1,
                             n_full // NS).astype(jnp.int32)

        def face_body(i, _):
            do_chunk((sid + i * NS) * F_CHUNK, F_CHUNK)
            return 0
        lax.fori_loop(0, n_mine_f, face_body, 0)

        if f_tail:
            @pl.when(sid == NS - 1)
            def _():
                do_chunk(n_full * F_CHUNK, f_tail)

        plsc.subcore_barrier()

        def wo_body(i, _):
            c = sid + i * NS
            v0 = c * V_CHUNK
            pltpu.sync_copy(acc_s.at[pl.ds(v0, V_CHUNK), :], zbuf)
            for bb in range(BH):
                b = cid * BH + bb
                for j in range(V_CHUNK // L):
                    sl = pl.ds(j * L, L)
                    r = iota + j * L
                    n0 = plsc.load_gather(zbuf, [r, cfull(bb * 3 + 0)])
                    n1 = plsc.load_gather(zbuf, [r, cfull(bb * 3 + 1)])
                    n2 = plsc.load_gather(zbuf, [r, cfull(bb * 3 + 2)])
                    s2 = n0 * n0 + n1 * n1 + n2 * n2
                    nrm = s2 * _rsqrt_newton(s2)
                    rec = 1.0 / jnp.maximum(nrm, 1e-6)
                    obuf[0, sl] = n0 * rec
                    obuf[1, sl] = n1 * rec
                    obuf[2, sl] = n2 * rec
                pltpu.sync_copy(obuf, out_hbm.at[:, b, pl.ds(v0, V_CHUNK)])
            return 0
        lax.fori_loop(0, n_mine, wo_body, 0)

    out_t = sc_kernel(vt, ft)
    return jnp.transpose(out_t, (1, 2, 0))


# speedup vs baseline: 5.5654x; 1.3362x over previous
import functools

import jax
import jax.numpy as jnp
from jax import lax
from jax.experimental import pallas as pl
from jax.experimental.pallas import tpu as pltpu
from jax.experimental.pallas import tpu_sc as plsc

V_CHUNK = 400
F_CHUNK = 128
RW = 8


def _rsqrt_newton(x):
    xu = plsc.bitcast(x, jnp.uint32)
    y = plsc.bitcast(jnp.uint32(0x5F3759DF) - (xu >> jnp.uint32(1)), jnp.float32)
    half = x * 0.5
    for _ in range(3):
        y = y * (1.5 - half * y * y)
    return y


def kernel(vertices, faces):
    faces = jnp.squeeze(faces).astype(jnp.int32)
    B, V, _ = vertices.shape
    F = faces.shape[0]
    vt = jnp.transpose(vertices, (2, 0, 1))
    ft = faces.T

    info = plsc.get_sparse_core_info()
    NC, NS, L = info.num_cores, info.num_subcores, info.num_lanes
    BH = B // NC
    n_full = F // F_CHUNK
    f_tail = F % F_CHUNK
    assert f_tail % L == 0
    nv_chunks = V // V_CHUNK
    mesh = plsc.VectorSubcoreMesh(core_axis_name="c", subcore_axis_name="s")

    @functools.partial(
        pl.kernel,
        out_type=jax.ShapeDtypeStruct((3, B, V), jnp.float32),
        mesh=mesh,
        compiler_params=pltpu.CompilerParams(
            needs_layout_passes=False, use_tc_tiling_on_sc=False),
        scratch_types=[
            pltpu.VMEM_SHARED((V, RW), jnp.float32),
            pltpu.VMEM_SHARED((V, RW), jnp.float32),
            pltpu.VMEM((3, V_CHUNK), jnp.float32),
            pltpu.VMEM((V_CHUNK, RW), jnp.float32),
            pltpu.VMEM((V_CHUNK, RW), jnp.float32),
            pltpu.VMEM((3, V_CHUNK), jnp.float32),
            pltpu.VMEM((2, 3, F_CHUNK), jnp.int32),
            pltpu.VMEM((2, 3, F_CHUNK, RW), jnp.float32),
            pltpu.VMEM((2, F_CHUNK, RW), jnp.float32),
            pltpu.SemaphoreType.DMA,
            pltpu.SemaphoreType.DMA,
            pltpu.SemaphoreType.DMA,
            pltpu.SemaphoreType.DMA,
            pltpu.SemaphoreType.DMA,
            pltpu.SemaphoreType.DMA,
        ],
    )
    def sc_kernel(verts_hbm, faces_hbm, out_hbm,
                  uw_s, acc_s, vbuf, uwbuf, zbuf, obuf, idxbuf, gbuf, sbuf,
                  sem_ia, sem_ib, sem_ga, sem_gb, sem_sa, sem_sb):
        cid = lax.axis_index("c")
        sid = lax.axis_index("s")
        iota = lax.iota(jnp.int32, L)
        zvec = jnp.zeros((L,), jnp.float32)

        def cfull(val):
            return jnp.full((L,), val, jnp.int32)

        for j in range(V_CHUNK // L):
            r = iota + j * L
            for col in range(RW):
                plsc.store_scatter(zbuf, [r, cfull(col)], zvec)

        n_mine = jnp.where(sid < (nv_chunks % NS), nv_chunks // NS + 1,
                           nv_chunks // NS).astype(jnp.int32)

        def build_body(i, _):
            c = sid + i * NS
            v0 = c * V_CHUNK
            for bb in range(BH):
                b = cid * BH + bb
                pltpu.sync_copy(verts_hbm.at[:, b, pl.ds(v0, V_CHUNK)], vbuf)

                def grp(j, _):
                    sl = pl.ds(j * L, L)
                    r = iota + j * L
                    x = vbuf[0, sl]
                    y = vbuf[1, sl]
                    z = vbuf[2, sl]
                    plsc.store_scatter(uwbuf, [r, cfull(bb)], y - x)
                    plsc.store_scatter(uwbuf, [r, cfull(BH + bb)], z - x)
                    return 0
                lax.fori_loop(0, V_CHUNK // L, grp, 0)
            pltpu.sync_copy(uwbuf, uw_s.at[pl.ds(v0, V_CHUNK), :])
            pltpu.sync_copy(zbuf, acc_s.at[pl.ds(v0, V_CHUNK), :])
            return 0
        lax.fori_loop(0, n_mine, build_body, 0)

        plsc.subcore_barrier()

        def compute(slot, ngrp):
            for j in range(ngrp):
                r = iota + j * L
                for bb in range(BH):
                    a0 = plsc.load_gather(gbuf, [cfull(slot), cfull(0), r, cfull(bb)])
                    a1 = plsc.load_gather(gbuf, [cfull(slot), cfull(1), r, cfull(bb)])
                    a2 = plsc.load_gather(gbuf, [cfull(slot), cfull(2), r, cfull(bb)])
                    c0 = plsc.load_gather(gbuf, [cfull(slot), cfull(0), r, cfull(BH + bb)])
                    c1 = plsc.load_gather(gbuf, [cfull(slot), cfull(1), r, cfull(BH + bb)])
                    c2 = plsc.load_gather(gbuf, [cfull(slot), cfull(2), r, cfull(BH + bb)])
                    plsc.store_scatter(sbuf, [cfull(slot), r, cfull(bb * 3 + 0)],
                                       a1 * c2 - a2 * c1)
                    plsc.store_scatter(sbuf, [cfull(slot), r, cfull(bb * 3 + 1)],
                                       a2 * c0 - a0 * c2)
                    plsc.store_scatter(sbuf, [cfull(slot), r, cfull(bb * 3 + 2)],
                                       a0 * c1 - a1 * c0)
                plsc.store_scatter(sbuf, [cfull(slot), r, cfull(6)], zvec)
                plsc.store_scatter(sbuf, [cfull(slot), r, cfull(7)], zvec)

        def fire_gathers(slot, sem):
            return [pltpu.async_copy(uw_s.at[idxbuf.at[slot, k]],
                                     gbuf.at[slot, k], sem)
                    for k in range(3)]

        def fire_scatters(slot, sem):
            return [pltpu.async_copy(sbuf.at[slot],
                                     acc_s.at[idxbuf.at[slot, k]], sem,
                                     add=True)
                    for k in range(3)]

        n_pairs = n_full // 2
        n_mine_p = jnp.where(sid < (n_pairs % NS), n_pairs // NS + 1,
                             n_pairs // NS).astype(jnp.int32)

        def pair_body(i, _):
            f0 = (sid + i * NS) * 2 * F_CHUNK
            da = pltpu.async_copy(faces_hbm.at[:, pl.ds(f0, F_CHUNK)],
                                  idxbuf.at[0], sem_ia)
            db = pltpu.async_copy(faces_hbm.at[:, pl.ds(f0 + F_CHUNK, F_CHUNK)],
                                  idxbuf.at[1], sem_ib)
            da.wait()
            ga = fire_gathers(0, sem_ga)
            db.wait()
            gb = fire_gathers(1, sem_gb)
            for d in ga:
                d.wait()
            compute(0, F_CHUNK // L)
            sa = fire_scatters(0, sem_sa)
            for d in gb:
                d.wait()
            compute(1, F_CHUNK // L)
            sb = fire_scatters(1, sem_sb)
            for d in sa:
                d.wait()
            for d in sb:
                d.wait()
            return 0
        lax.fori_loop(0, n_mine_p, pair_body, 0)

        def do_chunk_sync(f0, n):
            pltpu.sync_copy(faces_hbm.at[:, pl.ds(f0, n)],
                            idxbuf.at[0, :, pl.ds(0, n)])
            for k in range(3):
                pltpu.sync_copy(uw_s.at[idxbuf.at[0, k, pl.ds(0, n)]],
                                gbuf.at[0, k, pl.ds(0, n), :])
            compute(0, n // L)
            for k in range(3):
                pltpu.sync_copy(sbuf.at[0, pl.ds(0, n), :],
                                acc_s.at[idxbuf.at[0, k, pl.ds(0, n)]], add=True)

        if n_full % 2:
            @pl.when(sid == 0)
            def _():
                do_chunk_sync((n_full - 1) * F_CHUNK, F_CHUNK)

        if f_tail:
            @pl.when(sid == NS - 1)
            def _():
                do_chunk_sync(n_full * F_CHUNK, f_tail)

        plsc.subcore_barrier()

        def wo_body(i, _):
            c = sid + i * NS
            v0 = c * V_CHUNK
            pltpu.sync_copy(acc_s.at[pl.ds(v0, V_CHUNK), :], zbuf)
            for bb in range(BH):
                b = cid * BH + bb
                for j in range(V_CHUNK // L):
                    sl = pl.ds(j * L, L)
                    r = iota + j * L
                    n0 = plsc.load_gather(zbuf, [r, cfull(bb * 3 + 0)])
                    n1 = plsc.load_gather(zbuf, [r, cfull(bb * 3 + 1)])
                    n2 = plsc.load_gather(zbuf, [r, cfull(bb * 3 + 2)])
                    s2 = n0 * n0 + n1 * n1 + n2 * n2
                    nrm = s2 * _rsqrt_newton(s2)
                    rec = 1.0 / jnp.maximum(nrm, 1e-6)
                    obuf[0, sl] = n0 * rec
                    obuf[1, sl] = n1 * rec
                    obuf[2, sl] = n2 * rec
                pltpu.sync_copy(obuf, out_hbm.at[:, b, pl.ds(v0, V_CHUNK)])
            return 0
        lax.fori_loop(0, n_mine, wo_body, 0)

    out_t = sc_kernel(vt, ft)
    return jnp.transpose(out_t, (1, 2, 0))


# speedup vs baseline: 7.6362x; 1.3721x over previous
import functools

import jax
import jax.numpy as jnp
from jax import lax
from jax.experimental import pallas as pl
from jax.experimental.pallas import tpu as pltpu
from jax.experimental.pallas import tpu_sc as plsc

V_CHUNK = 400
F_CHUNK = 128
RW = 8


def _rsqrt_newton(x):
    xu = plsc.bitcast(x, jnp.uint32)
    y = plsc.bitcast(jnp.uint32(0x5F3759DF) - (xu >> jnp.uint32(1)), jnp.float32)
    half = x * 0.5
    for _ in range(3):
        y = y * (1.5 - half * y * y)
    return y


def kernel(vertices, faces):
    faces = jnp.squeeze(faces).astype(jnp.int32)
    B, V, _ = vertices.shape
    F = faces.shape[0]
    vt = jnp.transpose(vertices, (2, 0, 1))
    ft = faces.T

    info = plsc.get_sparse_core_info()
    NC, NS, L = info.num_cores, info.num_subcores, info.num_lanes
    BH = B // NC
    n_full = F // F_CHUNK
    n_quads = n_full // 4
    n_rem = n_full % 4
    f_tail = F % F_CHUNK
    assert f_tail % L == 0
    nv_chunks = V // V_CHUNK
    nv_pairs = nv_chunks // 2
    nv_rem = nv_chunks % 2
    mesh = plsc.VectorSubcoreMesh(core_axis_name="c", subcore_axis_name="s")

    @functools.partial(
        pl.kernel,
        out_type=jax.ShapeDtypeStruct((3, B, V), jnp.float32),
        mesh=mesh,
        compiler_params=pltpu.CompilerParams(
            needs_layout_passes=False, use_tc_tiling_on_sc=False),
        scratch_types=[
            pltpu.VMEM_SHARED((V, RW), jnp.float32),
            pltpu.VMEM_SHARED((V, RW), jnp.float32),
            pltpu.VMEM((2, 3, V_CHUNK), jnp.float32),
            pltpu.VMEM((V_CHUNK, RW), jnp.float32),
            pltpu.VMEM((2, V_CHUNK, RW), jnp.float32),
            pltpu.VMEM((2, 3, V_CHUNK), jnp.float32),
            pltpu.VMEM((4, 3, F_CHUNK), jnp.int32),
            pltpu.VMEM((3, 3, F_CHUNK, RW), jnp.float32),
            pltpu.VMEM((4, F_CHUNK, RW), jnp.float32),
        ] + [pltpu.SemaphoreType.DMA] * 19,
    )
    def sc_kernel(verts_hbm, faces_hbm, out_hbm,
                  uw_s, acc_s, vbuf, uwbuf, zbuf, obuf, idxbuf, gbuf, sbuf,
                  sem_b0, sem_b1, sem_p1w,
                  sem_i0, sem_i1, sem_i2, sem_i3,
                  sem_g0, sem_g1, sem_g2, sem_g3,
                  sem_s0, sem_s1, sem_s2, sem_s3,
                  sem_ra, sem_rb, sem_wa, sem_wb):
        cid = lax.axis_index("c")
        sid = lax.axis_index("s")
        iota = lax.iota(jnp.int32, L)
        zvec = jnp.zeros((L,), jnp.float32)
        sem_i = [sem_i0, sem_i1, sem_i2, sem_i3]
        sem_g = [sem_g0, sem_g1, sem_g2, sem_g3]
        sem_s = [sem_s0, sem_s1, sem_s2, sem_s3]

        def cfull(val):
            return jnp.full((L,), val, jnp.int32)

        for j in range(V_CHUNK // L):
            r = iota + j * L
            for col in range(RW):
                plsc.store_scatter(zbuf, [cfull(0), r, cfull(col)], zvec)
        for t in range(4):
            for j in range(F_CHUNK // L):
                r = iota + j * L
                plsc.store_scatter(sbuf, [cfull(t), r, cfull(6)], zvec)
                plsc.store_scatter(sbuf, [cfull(t), r, cfull(7)], zvec)

        n_mine = jnp.where(sid < (nv_chunks % NS), nv_chunks // NS + 1,
                           nv_chunks // NS).astype(jnp.int32)

        def build_body(i, _):
            c = sid + i * NS
            v0 = c * V_CHUNK
            r0 = pltpu.async_copy(
                verts_hbm.at[:, cid * BH + 0, pl.ds(v0, V_CHUNK)],
                vbuf.at[0], sem_b0)
            r1 = pltpu.async_copy(
                verts_hbm.at[:, cid * BH + 1, pl.ds(v0, V_CHUNK)],
                vbuf.at[1], sem_b1)
            for bb, rd in ((0, r0), (1, r1)):
                rd.wait()

                def grp(j, _):
                    sl = pl.ds(j * L, L)
                    r = iota + j * L
                    x = vbuf[bb, 0, sl]
                    y = vbuf[bb, 1, sl]
                    z = vbuf[bb, 2, sl]
                    plsc.store_scatter(uwbuf, [r, cfull(bb)], y - x)
                    plsc.store_scatter(uwbuf, [r, cfull(BH + bb)], z - x)
                    return 0
                lax.fori_loop(0, V_CHUNK // L, grp, 0)
            w0 = pltpu.async_copy(uwbuf, uw_s.at[pl.ds(v0, V_CHUNK), :],
                                  sem_p1w)
            w1 = pltpu.async_copy(zbuf.at[0], acc_s.at[pl.ds(v0, V_CHUNK), :],
                                  sem_p1w)
            w0.wait()
            w1.wait()
            return 0
        lax.fori_loop(0, n_mine, build_body, 0)

        plsc.subcore_barrier()

        def compute(gslot, sslot, ngrp):
            def cgrp(j, _):
                r = iota + j * L
                for bb in range(BH):
                    a0 = plsc.load_gather(gbuf, [cfull(gslot), cfull(0), r, cfull(bb)])
                    a1 = plsc.load_gather(gbuf, [cfull(gslot), cfull(1), r, cfull(bb)])
                    a2 = plsc.load_gather(gbuf, [cfull(gslot), cfull(2), r, cfull(bb)])
                    c0 = plsc.load_gather(gbuf, [cfull(gslot), cfull(0), r, cfull(BH + bb)])
                    c1 = plsc.load_gather(gbuf, [cfull(gslot), cfull(1), r, cfull(BH + bb)])
                    c2 = plsc.load_gather(gbuf, [cfull(gslot), cfull(2), r, cfull(BH + bb)])
                    plsc.store_scatter(sbuf, [cfull(sslot), r, cfull(bb * 3 + 0)],
                                       a1 * c2 - a2 * c1)
                    plsc.store_scatter(sbuf, [cfull(sslot), r, cfull(bb * 3 + 1)],
                                       a2 * c0 - a0 * c2)
                    plsc.store_scatter(sbuf, [cfull(sslot), r, cfull(bb * 3 + 2)],
                                       a0 * c1 - a1 * c0)
                return 0
            lax.fori_loop(0, ngrp, cgrp, 0)

        def fire_gathers(islot, gslot, sem):
            return [pltpu.async_copy(uw_s.at[idxbuf.at[islot, k]],
                                     gbuf.at[gslot, k], sem)
                    for k in range(3)]

        def fire_scatters(sslot, islot, sem):
            return [pltpu.async_copy(sbuf.at[sslot],
                                     acc_s.at[idxbuf.at[islot, k]], sem,
                                     add=True)
                    for k in range(3)]

        n_mine_q = jnp.where(sid < (n_quads % NS), n_quads // NS + 1,
                             n_quads // NS).astype(jnp.int32)

        def quad_body(i, _):
            f0 = (sid + i * NS) * 4 * F_CHUNK
            di = [pltpu.async_copy(
                      faces_hbm.at[:, pl.ds(f0 + t * F_CHUNK, F_CHUNK)],
                      idxbuf.at[t], sem_i[t]) for t in range(4)]
            di[0].wait()
            g0 = fire_gathers(0, 0, sem_g[0])
            di[1].wait()
            g1 = fire_gathers(1, 1, sem_g[1])
            for d in g0:
                d.wait()
            compute(0, 0, F_CHUNK // L)
            s0 = fire_scatters(0, 0, sem_s[0])
            di[2].wait()
            g2 = fire_gathers(2, 2, sem_g[2])
            for d in g1:
                d.wait()
            compute(1, 1, F_CHUNK // L)
            s1 = fire_scatters(1, 1, sem_s[1])
            di[3].wait()
            g3 = fire_gathers(3, 0, sem_g[3])
            for d in g2:
                d.wait()
            compute(2, 2, F_CHUNK // L)
            s2 = fire_scatters(2, 2, sem_s[2])
            for d in g3:
                d.wait()
            compute(0, 3, F_CHUNK // L)
            s3 = fire_scatters(3, 3, sem_s[3])
            for s in (s0, s1, s2, s3):
                for d in s:
                    d.wait()
            return 0
        lax.fori_loop(0, n_mine_q, quad_body, 0)

        def do_chunk_sync(f0, n):
            pltpu.sync_copy(faces_hbm.at[:, pl.ds(f0, n)],
                            idxbuf.at[0, :, pl.ds(0, n)])
            for k in range(3):
                pltpu.sync_copy(uw_s.at[idxbuf.at[0, k, pl.ds(0, n)]],
                                gbuf.at[0, k, pl.ds(0, n), :])
            compute(0, 0, n // L)
            for k in range(3):
                pltpu.sync_copy(sbuf.at[0, pl.ds(0, n), :],
                                acc_s.at[idxbuf.at[0, k, pl.ds(0, n)]], add=True)

        for t in range(n_rem):
            @pl.when(sid == t)
            def _(t=t):
                do_chunk_sync((n_quads * 4 + t) * F_CHUNK, F_CHUNK)

        if f_tail:
            @pl.when(sid == NS - 1)
            def _():
                do_chunk_sync(n_full * F_CHUNK, f_tail)

        plsc.subcore_barrier()

        def norm_chunk(zslot, bb, oslot):
            def ngrp_body(j, _):
                sl = pl.ds(j * L, L)
                r = iota + j * L
                n0 = plsc.load_gather(zbuf, [cfull(zslot), r, cfull(bb * 3 + 0)])
                n1 = plsc.load_gather(zbuf, [cfull(zslot), r, cfull(bb * 3 + 1)])
                n2 = plsc.load_gather(zbuf, [cfull(zslot), r, cfull(bb * 3 + 2)])
                s2 = n0 * n0 + n1 * n1 + n2 * n2
                nrm = s2 * _rsqrt_newton(s2)
                rec = 1.0 / jnp.maximum(nrm, 1e-6)
                obuf[oslot, 0, sl] = n0 * rec
                obuf[oslot, 1, sl] = n1 * rec
                obuf[oslot, 2, sl] = n2 * rec
                return 0
            lax.fori_loop(0, V_CHUNK // L, ngrp_body, 0)

        def out_write(oslot, bb, v0, sem):
            return pltpu.async_copy(
                obuf.at[oslot],
                out_hbm.at[:, cid * BH + bb, pl.ds(v0, V_CHUNK)], sem)

        n_mine_p3 = jnp.where(sid < (nv_pairs % NS), nv_pairs // NS + 1,
                              nv_pairs // NS).astype(jnp.int32)

        def p3_body(i, _):
            p = sid + i * NS
            va = (2 * p) * V_CHUNK
            vb = (2 * p + 1) * V_CHUNK
            ra = pltpu.async_copy(acc_s.at[pl.ds(va, V_CHUNK), :],
                                  zbuf.at[0], sem_ra)
            rb = pltpu.async_copy(acc_s.at[pl.ds(vb, V_CHUNK), :],
                                  zbuf.at[1], sem_rb)
            ra.wait()
            norm_chunk(0, 0, 0)
            w00 = out_write(0, 0, va, sem_wa)
            norm_chunk(0, 1, 1)
            w01 = out_write(1, 1, va, sem_wa)
            rb.wait()
            w00.wait()
            norm_chunk(1, 0, 0)
            w10 = out_write(0, 0, vb, sem_wb)
            w01.wait()
            norm_chunk(1, 1, 1)
            w11 = out_write(1, 1, vb, sem_wb)
            w10.wait()
            w11.wait()
            return 0
        lax.fori_loop(0, n_mine_p3, p3_body, 0)

        if nv_rem:
            @pl.when(sid == 0)
            def _():
                v0 = (nv_chunks - 1) * V_CHUNK
                pltpu.sync_copy(acc_s.at[pl.ds(v0, V_CHUNK), :], zbuf.at[0])
                for bb in range(BH):
                    norm_chunk(0, bb, 0)
                    out_write(0, bb, v0, sem_wa).wait()

    out_t = sc_kernel(vt, ft)
    return jnp.transpose(out_t, (1, 2, 0))


# speedup vs baseline: 8.2038x; 1.0743x over previous
import functools

import jax
import jax.numpy as jnp
from jax import lax
from jax.experimental import pallas as pl
from jax.experimental.pallas import tpu as pltpu
from jax.experimental.pallas import tpu_sc as plsc

V_CHUNK = 400
F_CHUNK = 128
RW = 8


def _rsqrt_newton(x):
    xu = plsc.bitcast(x, jnp.uint32)
    y = plsc.bitcast(jnp.uint32(0x5F3759DF) - (xu >> jnp.uint32(1)), jnp.float32)
    half = x * 0.5
    for _ in range(3):
        y = y * (1.5 - half * y * y)
    return y


def kernel(vertices, faces):
    faces = jnp.squeeze(faces).astype(jnp.int32)
    B, V, _ = vertices.shape
    F = faces.shape[0]
    vt = jnp.transpose(vertices, (2, 0, 1))
    ft = faces.T

    info = plsc.get_sparse_core_info()
    NC, NS, L = info.num_cores, info.num_subcores, info.num_lanes
    BH = B // NC
    n_full = F // F_CHUNK
    BLK = 8
    n_blocks = n_full // BLK
    n_rem = n_full % BLK
    f_tail = F % F_CHUNK
    assert f_tail % L == 0
    nv_chunks = V // V_CHUNK
    nv_pairs = nv_chunks // 2
    nv_rem = nv_chunks % 2
    mesh = plsc.VectorSubcoreMesh(core_axis_name="c", subcore_axis_name="s")

    @functools.partial(
        pl.kernel,
        out_type=jax.ShapeDtypeStruct((3, B, V), jnp.float32),
        mesh=mesh,
        compiler_params=pltpu.CompilerParams(
            needs_layout_passes=False, use_tc_tiling_on_sc=False),
        scratch_types=[
            pltpu.VMEM_SHARED((V, RW), jnp.float32),
            pltpu.VMEM_SHARED((V, RW), jnp.float32),
            pltpu.VMEM((2, 3, V_CHUNK), jnp.float32),
            pltpu.VMEM((V_CHUNK, RW), jnp.float32),
            pltpu.VMEM((2, V_CHUNK, RW), jnp.float32),
            pltpu.VMEM((2, 3, V_CHUNK), jnp.float32),
            pltpu.VMEM((8, 3, F_CHUNK), jnp.int32),
            pltpu.VMEM((2, 3, F_CHUNK, RW), jnp.float32),
            pltpu.VMEM((4, F_CHUNK, RW), jnp.float32),
        ] + [pltpu.SemaphoreType.DMA] * 21,
    )
    def sc_kernel(verts_hbm, faces_hbm, out_hbm,
                  uw_s, acc_s, vbuf, uwbuf, zbuf, obuf, idxbuf, gbuf, sbuf,
                  sem_b0, sem_b1, sem_p1w,
                  sem_i0, sem_i1, sem_i2, sem_i3,
                  sem_i4, sem_i5, sem_i6, sem_i7,
                  sem_g0, sem_g1,
                  sem_s0, sem_s1, sem_s2, sem_s3,
                  sem_ra, sem_rb, sem_wa, sem_wb):
        cid = lax.axis_index("c")
        sid = lax.axis_index("s")
        iota = lax.iota(jnp.int32, L)
        zvec = jnp.zeros((L,), jnp.float32)
        sem_i = [sem_i0, sem_i1, sem_i2, sem_i3, sem_i4, sem_i5, sem_i6, sem_i7]
        sem_g = [sem_g0, sem_g1]
        sem_s = [sem_s0, sem_s1, sem_s2, sem_s3]

        def cfull(val):
            return jnp.full((L,), val, jnp.int32)

        for j in range(V_CHUNK // L):
            r = iota + j * L
            for col in range(RW):
                plsc.store_scatter(zbuf, [cfull(0), r, cfull(col)], zvec)
        for t in range(4):
            for j in range(F_CHUNK // L):
                r = iota + j * L
                plsc.store_scatter(sbuf, [cfull(t), r, cfull(6)], zvec)
                plsc.store_scatter(sbuf, [cfull(t), r, cfull(7)], zvec)

        n_mine = jnp.where(sid < (nv_chunks % NS), nv_chunks // NS + 1,
                           nv_chunks // NS).astype(jnp.int32)

        def build_body(i, _):
            c = sid + i * NS
            v0 = c * V_CHUNK
            r0 = pltpu.async_copy(
                verts_hbm.at[:, cid * BH + 0, pl.ds(v0, V_CHUNK)],
                vbuf.at[0], sem_b0)
            r1 = pltpu.async_copy(
                verts_hbm.at[:, cid * BH + 1, pl.ds(v0, V_CHUNK)],
                vbuf.at[1], sem_b1)
            for bb, rd in ((0, r0), (1, r1)):
                rd.wait()

                def grp(j, _):
                    sl = pl.ds(j * L, L)
                    r = iota + j * L
                    x = vbuf[bb, 0, sl]
                    y = vbuf[bb, 1, sl]
                    z = vbuf[bb, 2, sl]
                    plsc.store_scatter(uwbuf, [r, cfull(bb)], y - x)
                    plsc.store_scatter(uwbuf, [r, cfull(BH + bb)], z - x)
                    return 0
                lax.fori_loop(0, V_CHUNK // L, grp, 0)
            w0 = pltpu.async_copy(uwbuf, uw_s.at[pl.ds(v0, V_CHUNK), :],
                                  sem_p1w)
            w1 = pltpu.async_copy(zbuf.at[0], acc_s.at[pl.ds(v0, V_CHUNK), :],
                                  sem_p1w)
            w0.wait()
            w1.wait()
            return 0
        lax.fori_loop(0, n_mine, build_body, 0)

        plsc.subcore_barrier()

        def compute(gslot, sslot, ngrp):
            def cgrp(j, _):
                r = iota + j * L
                for bb in range(BH):
                    a0 = plsc.load_gather(gbuf, [cfull(gslot), cfull(0), r, cfull(bb)])
                    a1 = plsc.load_gather(gbuf, [cfull(gslot), cfull(1), r, cfull(bb)])
                    a2 = plsc.load_gather(gbuf, [cfull(gslot), cfull(2), r, cfull(bb)])
                    c0 = plsc.load_gather(gbuf, [cfull(gslot), cfull(0), r, cfull(BH + bb)])
                    c1 = plsc.load_gather(gbuf, [cfull(gslot), cfull(1), r, cfull(BH + bb)])
                    c2 = plsc.load_gather(gbuf, [cfull(gslot), cfull(2), r, cfull(BH + bb)])
                    plsc.store_scatter(sbuf, [cfull(sslot), r, cfull(bb * 3 + 0)],
                                       a1 * c2 - a2 * c1)
                    plsc.store_scatter(sbuf, [cfull(sslot), r, cfull(bb * 3 + 1)],
                                       a2 * c0 - a0 * c2)
                    plsc.store_scatter(sbuf, [cfull(sslot), r, cfull(bb * 3 + 2)],
                                       a0 * c1 - a1 * c0)
                return 0
            lax.fori_loop(0, ngrp, cgrp, 0)

        def fire_gathers(islot, gslot, sem):
            return [pltpu.async_copy(uw_s.at[idxbuf.at[islot, k]],
                                     gbuf.at[gslot, k], sem)
                    for k in range(3)]

        def fire_scatters(sslot, islot, sem):
            return [pltpu.async_copy(sbuf.at[sslot],
                                     acc_s.at[idxbuf.at[islot, k]], sem,
                                     add=True)
                    for k in range(3)]

        n_mine_q = jnp.where(sid < (n_blocks % NS), n_blocks // NS + 1,
                             n_blocks // NS).astype(jnp.int32)

        def blk_body(i, _):
            f0 = (sid + i * NS) * BLK * F_CHUNK
            di = [pltpu.async_copy(
                      faces_hbm.at[:, pl.ds(f0 + t * F_CHUNK, F_CHUNK)],
                      idxbuf.at[t], sem_i[t]) for t in range(BLK)]
            di[0].wait()
            g_cur = fire_gathers(0, 0, sem_g[0])
            s_desc = [None] * BLK
            for t in range(BLK):
                if t + 1 < BLK:
                    di[t + 1].wait()
                    g_next = fire_gathers(t + 1, (t + 1) % 2, sem_g[(t + 1) % 2])
                if t >= 4:
                    for d in s_desc[t - 4]:
                        d.wait()
                for d in g_cur:
                    d.wait()
                compute(t % 2, t % 4, F_CHUNK // L)
                s_desc[t] = fire_scatters(t % 4, t, sem_s[t % 4])
                if t + 1 < BLK:
                    g_cur = g_next
            for t in range(BLK - 4, BLK):
                for d in s_desc[t]:
                    d.wait()
            return 0
        lax.fori_loop(0, n_mine_q, blk_body, 0)

        def do_chunk_sync(f0, n):
            pltpu.sync_copy(faces_hbm.at[:, pl.ds(f0, n)],
                            idxbuf.at[0, :, pl.ds(0, n)])
            for k in range(3):
                pltpu.sync_copy(uw_s.at[idxbuf.at[0, k, pl.ds(0, n)]],
                                gbuf.at[0, k, pl.ds(0, n), :])
            compute(0, 0, n // L)
            for k in range(3):
                pltpu.sync_copy(sbuf.at[0, pl.ds(0, n), :],
                                acc_s.at[idxbuf.at[0, k, pl.ds(0, n)]], add=True)

        for t in range(n_rem):
            @pl.when(sid == t)
            def _(t=t):
                do_chunk_sync((n_blocks * BLK + t) * F_CHUNK, F_CHUNK)

        if f_tail:
            @pl.when(sid == NS - 1)
            def _():
                do_chunk_sync(n_full * F_CHUNK, f_tail)

        plsc.subcore_barrier()

        def norm_chunk(zslot, bb, oslot):
            def ngrp_body(j, _):
                sl = pl.ds(j * L, L)
                r = iota + j * L
                n0 = plsc.load_gather(zbuf, [cfull(zslot), r, cfull(bb * 3 + 0)])
                n1 = plsc.load_gather(zbuf, [cfull(zslot), r, cfull(bb * 3 + 1)])
                n2 = plsc.load_gather(zbuf, [cfull(zslot), r, cfull(bb * 3 + 2)])
                s2 = n0 * n0 + n1 * n1 + n2 * n2
                nrm = s2 * _rsqrt_newton(s2)
                rec = 1.0 / jnp.maximum(nrm, 1e-6)
                obuf[oslot, 0, sl] = n0 * rec
                obuf[oslot, 1, sl] = n1 * rec
                obuf[oslot, 2, sl] = n2 * rec
                return 0
            lax.fori_loop(0, V_CHUNK // L, ngrp_body, 0)

        def out_write(oslot, bb, v0, sem):
            return pltpu.async_copy(
                obuf.at[oslot],
                out_hbm.at[:, cid * BH + bb, pl.ds(v0, V_CHUNK)], sem)

        n_mine_p3 = jnp.where(sid < (nv_pairs % NS), nv_pairs // NS + 1,
                              nv_pairs // NS).astype(jnp.int32)

        def p3_body(i, _):
            p = sid + i * NS
            va = (2 * p) * V_CHUNK
            vb = (2 * p + 1) * V_CHUNK
            ra = pltpu.async_copy(acc_s.at[pl.ds(va, V_CHUNK), :],
                                  zbuf.at[0], sem_ra)
            rb = pltpu.async_copy(acc_s.at[pl.ds(vb, V_CHUNK), :],
                                  zbuf.at[1], sem_rb)
            ra.wait()
            norm_chunk(0, 0, 0)
            w00 = out_write(0, 0, va, sem_wa)
            norm_chunk(0, 1, 1)
            w01 = out_write(1, 1, va, sem_wa)
            rb.wait()
            w00.wait()
            norm_chunk(1, 0, 0)
            w10 = out_write(0, 0, vb, sem_wb)
            w01.wait()
            norm_chunk(1, 1, 1)
            w11 = out_write(1, 1, vb, sem_wb)
            w10.wait()
            w11.wait()
            return 0
        lax.fori_loop(0, n_mine_p3, p3_body, 0)

        if nv_rem:
            @pl.when(sid == 0)
            def _():
                v0 = (nv_chunks - 1) * V_CHUNK
                pltpu.sync_copy(acc_s.at[pl.ds(v0, V_CHUNK), :], zbuf.at[0])
                for bb in range(BH):
                    norm_chunk(0, bb, 0)
                    out_write(0, bb, v0, sem_wa).wait()

    out_t = sc_kernel(vt, ft)
    return jnp.transpose(out_t, (1, 2, 0))
